# baseline (speedup 1.0000x reference)
import functools

import jax
import jax.numpy as jnp
from jax import lax
from jax.experimental import pallas as pl
from jax.experimental.pallas import tpu as pltpu
from jax.experimental.pallas import tpu_sc as plsc

N = 10000
E = 320000
D = 128
H = 128
C = 10
G = 64

NC = 2
NS = 16
L = 16

CHUNK = 80
K = 4
EPT = 10240
E_PAD = NC * NS * EPT
NCHUNK = EPT // CHUNK
NPAD = 10240
DW = 16
E_PAD2 = E_PAD + 1024

_mesh = plsc.VectorSubcoreMesh(
    core_axis_name="c", subcore_axis_name="s", num_cores=NC, num_subcores=NS)


@functools.partial(
    pl.kernel,
    out_type=jax.ShapeDtypeStruct((NC, NPAD, DW), jnp.float32),
    mesh=_mesh,
    scratch_types=[
        pltpu.VMEM((CHUNK,), jnp.int32),
        pltpu.VMEM((CHUNK,), jnp.int32),
        pltpu.VMEM((CHUNK, DW), jnp.float32),
        pltpu.VMEM_SHARED((NPAD, DW), jnp.float32),
        pltpu.SemaphoreType.DMA,
        pltpu.SemaphoreType.DMA,
    ],
)
def _sc_degree(dst_hbm, out_hbm, d0_v, d1_v, rows_v, acc_sh, dsem0, dsem1):
    c = lax.axis_index("c")
    s = lax.axis_index("s")

    one16 = jnp.ones((L,), jnp.float32)
    zero16 = jnp.zeros((L,), jnp.float32)
    zrows = NPAD // NS

    @pl.loop(0, CHUNK)
    def _(r):
        rows_v[r] = zero16

    @pl.loop(0, zrows // CHUNK)
    def _(j):
        pltpu.sync_copy(rows_v, acc_sh.at[pl.ds(s * zrows + j * CHUNK, CHUNK)])

    @pl.loop(0, CHUNK)
    def _(r):
        rows_v[r] = one16

    base = (c * NS + s) * NCHUNK

    plsc.subcore_barrier()

    @pl.loop(0, NCHUNK)
    def _(j):
        pltpu.sync_copy(dst_hbm.at[pl.ds((base + j) * CHUNK, CHUNK)], d0_v)
        pltpu.sync_copy(rows_v, acc_sh.at[d0_v], add=True)

    plsc.subcore_barrier()

    @pl.loop(0, zrows // CHUNK)
    def _(j):
        rb = s * zrows + j * CHUNK
        pltpu.sync_copy(acc_sh.at[pl.ds(rb, CHUNK)], rows_v)
        pltpu.sync_copy(rows_v, out_hbm.at[c, pl.ds(rb, CHUNK)])


@functools.partial(
    pl.kernel,
    out_type=jax.ShapeDtypeStruct((NC, NPAD, D), jnp.float32),
    mesh=_mesh,
    scratch_types=(
        [pltpu.VMEM((CHUNK,), jnp.int32) for _ in range(K)]
        + [pltpu.VMEM((CHUNK,), jnp.int32) for _ in range(K)]
        + [pltpu.VMEM((CHUNK, D), jnp.float32) for _ in range(K)]
        + [pltpu.VMEM_SHARED((NPAD, D), jnp.float32)]
    ),
)
def _sc_segsum(g_hbm, src_hbm, dst_hbm, out_hbm, *scr):
    s_bufs = scr[:K]
    d_bufs = scr[K:2 * K]
    r_bufs = scr[2 * K:3 * K]
    acc_sh = scr[3 * K]
    rows0_v = r_bufs[0]

    c = lax.axis_index("c")
    s = lax.axis_index("s")

    zero16 = jnp.zeros((L,), jnp.float32)

    @pl.loop(0, CHUNK)
    def _(r):
        @pl.loop(0, D // L)
        def _(t):
            rows0_v[r, pl.ds(t * L, L)] = zero16

    zrows = NPAD // NS

    @pl.loop(0, zrows // CHUNK)
    def _(j):
        pltpu.sync_copy(rows0_v, acc_sh.at[pl.ds(s * zrows + j * CHUNK, CHUNK)])

    base = (c * NS + s) * NCHUNK

    def sidx(chunk, buf, sem):
        return pltpu.make_async_copy(
            src_hbm.at[pl.ds((base + chunk) * CHUNK, CHUNK)], buf, sem)

    def didx(chunk, buf, sem):
        return pltpu.make_async_copy(
            dst_hbm.at[pl.ds((base + chunk) * CHUNK, CHUNK)], buf, sem)

    def gather(ibuf, rbuf, sem):
        return pltpu.make_async_copy(g_hbm.at[ibuf], rbuf, sem)

    plsc.subcore_barrier()

    def _edge_loop(*sems):
        gsems = sems[:K]
        ssems = sems[K:2 * K]
        dsems = sems[2 * K:]

        @pl.loop(0, NCHUNK, step=K)
        def _(j):
            for b in range(K):
                sidx(j + b, s_bufs[b], ssems[b]).start()
                didx(j + b, d_bufs[b], dsems[b]).start()
            for b in range(K):
                sidx(j + b, s_bufs[b], ssems[b]).wait()
                gather(s_bufs[b], r_bufs[b], gsems[b]).start()
            for b in range(K):
                didx(j + b, d_bufs[b], dsems[b]).wait()
            for b in range(K):
                gather(s_bufs[b], r_bufs[b], gsems[b]).wait()
            scats = []
            for b in range(K):
                sc = pltpu.make_async_copy(
                    r_bufs[b], acc_sh.at[d_bufs[b]], gsems[b])
                sc.start(add=True)
                scats.append(sc)
            for sc in scats:
                sc.wait()

    pl.run_scoped(_edge_loop, *([pltpu.SemaphoreType.DMA(())] * (3 * K)))

    plsc.subcore_barrier()

    @pl.loop(0, zrows // CHUNK)
    def _(j):
        ob = s * zrows + j * CHUNK
        pltpu.sync_copy(acc_sh.at[pl.ds(ob, CHUNK)], rows0_v)
        pltpu.sync_copy(rows0_v, out_hbm.at[c, pl.ds(ob, CHUNK)])


def _tc_first(deg_ref, x_ref, w1_ref, g_ref, dinv_ref):
    deg = 1.0 + deg_ref[0, 0:N, 0:1] + deg_ref[1, 0:N, 0:1]
    dinv = lax.rsqrt(deg)
    h = jnp.dot(x_ref[...], w1_ref[...], preferred_element_type=jnp.float32)
    g_ref[pl.ds(0, N), :] = dinv * h
    g_ref[pl.ds(N, NPAD - N), :] = jnp.zeros((NPAD - N, D), jnp.float32)
    dinv_ref[...] = dinv


def _tc_mid(s_ref, g_ref, dinv_ref, b_ref, w2_ref, g2_ref):
    dinv = dinv_ref[...]
    z = dinv * (s_ref[0, 0:N, :] + s_ref[1, 0:N, :] + g_ref[0:N, :]) + b_ref[...]
    z = jnp.maximum(z, 0.0)
    h = jnp.dot(z, w2_ref[...], preferred_element_type=jnp.float32)
    g2_ref[pl.ds(0, N), :] = dinv * h
    g2_ref[pl.ds(N, NPAD - N), :] = jnp.zeros((NPAD - N, D), jnp.float32)


def _tc_head(s_ref, g_ref, dinv_ref, b_ref, batch_ref, wfc_ref, bfc_ref, out_ref):
    dinv = dinv_ref[...]
    z = dinv * (s_ref[0, 0:N, :] + s_ref[1, 0:N, :] + g_ref[0:N, :]) + b_ref[...]
    z = jnp.maximum(z, 0.0)
    seg = lax.broadcasted_iota(jnp.int32, (G, N), 0)
    oh = (seg == batch_ref[...]).astype(jnp.float32)
    psum = jnp.dot(oh, z, preferred_element_type=jnp.float32)
    cnt = jnp.dot(oh, jnp.ones((N, 1), jnp.float32),
                  preferred_element_type=jnp.float32)
    pooled = psum / jnp.maximum(cnt, 1.0)
    out_ref[...] = jnp.dot(pooled, wfc_ref[...],
                           preferred_element_type=jnp.float32) + bfc_ref[...]


def kernel(x, edge_index, batch, W1, b1, W2, b2, Wfc, bfc):
    src = edge_index[0]
    dst = edge_index[1]
    pad = jnp.full((E_PAD2 - E,), N, jnp.int32)
    srcp = jnp.concatenate([src, pad])
    dstp = jnp.concatenate([dst, pad])
    batch_row = batch.reshape(1, N)

    degp = _sc_degree(dstp)

    g1, dinv = pl.pallas_call(
        _tc_first,
        out_shape=(jax.ShapeDtypeStruct((NPAD, D), jnp.float32),
                   jax.ShapeDtypeStruct((N, 1), jnp.float32)),
    )(degp, x, W1)

    s1 = _sc_segsum(g1, srcp, dstp)

    g2 = pl.pallas_call(
        _tc_mid,
        out_shape=jax.ShapeDtypeStruct((NPAD, D), jnp.float32),
    )(s1, g1, dinv, b1.reshape(1, H), W2)

    s2 = _sc_segsum(g2, srcp, dstp)

    out = pl.pallas_call(
        _tc_head,
        out_shape=jax.ShapeDtypeStruct((G, C), jnp.float32),
    )(s2, g2, dinv, b2.reshape(1, H), batch_row, Wfc, bfc.reshape(1, C))
    return out


# speedup vs baseline: 1.0752x; 1.0752x over previous
import functools

import jax
import jax.numpy as jnp
from jax import lax
from jax.experimental import pallas as pl
from jax.experimental.pallas import tpu as pltpu
from jax.experimental.pallas import tpu_sc as plsc

N = 10000
E = 320000
D = 128
H = 128
C = 10
G = 64

NC = 2
NS = 16
L = 16

CHUNK = 80
K = 4
EPT = 10240
E_PAD = NC * NS * EPT
NCHUNK = EPT // CHUNK
NPAD = 10240
DW = 16
E_PAD2 = E_PAD + 1024

_mesh = plsc.VectorSubcoreMesh(
    core_axis_name="c", subcore_axis_name="s", num_cores=NC, num_subcores=NS)


@functools.partial(
    pl.kernel,
    out_type=jax.ShapeDtypeStruct((NC * NPAD,), jnp.float32),
    mesh=_mesh,
    scratch_types=[
        pltpu.VMEM((CHUNK,), jnp.int32),
        pltpu.VMEM((CHUNK,), jnp.float32),
        pltpu.VMEM_SHARED((NPAD,), jnp.float32),
    ],
)
def _sc_degree(dst_hbm, out_hbm, d0_v, ones_v, acc_sh, ):
    c = lax.axis_index("c")
    s = lax.axis_index("s")

    one16 = jnp.ones((L,), jnp.float32)
    zero16 = jnp.zeros((L,), jnp.float32)
    zrows = NPAD // NS

    @pl.loop(0, CHUNK // L)
    def _(r):
        ones_v[pl.ds(r * L, L)] = zero16

    @pl.loop(0, zrows // CHUNK)
    def _(j):
        pltpu.sync_copy(ones_v, acc_sh.at[pl.ds(s * zrows + j * CHUNK, CHUNK)])

    @pl.loop(0, CHUNK // L)
    def _(r):
        ones_v[pl.ds(r * L, L)] = one16

    base = (c * NS + s) * NCHUNK

    plsc.subcore_barrier()

    @pl.loop(0, NCHUNK)
    def _(j):
        pltpu.sync_copy(dst_hbm.at[pl.ds((base + j) * CHUNK, CHUNK)], d0_v)
        pltpu.sync_copy(ones_v, acc_sh.at[d0_v], add=True)

    plsc.subcore_barrier()

    @pl.loop(0, zrows // CHUNK)
    def _(j):
        rb = s * zrows + j * CHUNK
        pltpu.sync_copy(acc_sh.at[pl.ds(rb, CHUNK)], ones_v)
        pltpu.sync_copy(ones_v, out_hbm.at[pl.ds(c * NPAD + rb, CHUNK)])


@functools.partial(
    pl.kernel,
    out_type=jax.ShapeDtypeStruct((NC, NPAD, D), jnp.float32),
    mesh=_mesh,
    scratch_types=(
        [pltpu.VMEM((CHUNK,), jnp.int32) for _ in range(K)]
        + [pltpu.VMEM((CHUNK,), jnp.int32) for _ in range(K)]
        + [pltpu.VMEM((CHUNK, D), jnp.float32) for _ in range(K)]
        + [pltpu.VMEM_SHARED((NPAD, D), jnp.float32)]
    ),
)
def _sc_segsum(g_hbm, src_hbm, dst_hbm, out_hbm, *scr):
    s_bufs = scr[:K]
    d_bufs = scr[K:2 * K]
    r_bufs = scr[2 * K:3 * K]
    acc_sh = scr[3 * K]
    rows0_v = r_bufs[0]

    c = lax.axis_index("c")
    s = lax.axis_index("s")

    zero16 = jnp.zeros((L,), jnp.float32)

    @pl.loop(0, CHUNK)
    def _(r):
        @pl.loop(0, D // L)
        def _(t):
            rows0_v[r, pl.ds(t * L, L)] = zero16

    zrows = NPAD // NS

    @pl.loop(0, zrows // CHUNK)
    def _(j):
        pltpu.sync_copy(rows0_v, acc_sh.at[pl.ds(s * zrows + j * CHUNK, CHUNK)])

    base = (c * NS + s) * NCHUNK

    def sidx(chunk, buf, sem):
        return pltpu.make_async_copy(
            src_hbm.at[pl.ds((base + chunk) * CHUNK, CHUNK)], buf, sem)

    def didx(chunk, buf, sem):
        return pltpu.make_async_copy(
            dst_hbm.at[pl.ds((base + chunk) * CHUNK, CHUNK)], buf, sem)

    def gather(ibuf, rbuf, sem):
        return pltpu.make_async_copy(g_hbm.at[ibuf], rbuf, sem)

    plsc.subcore_barrier()

    def _edge_loop(*sems):
        gsems = sems[:K]
        ssems = sems[K:2 * K]
        dsems = sems[2 * K:]

        @pl.loop(0, NCHUNK, step=K)
        def _(j):
            for b in range(K):
                sidx(j + b, s_bufs[b], ssems[b]).start()
                didx(j + b, d_bufs[b], dsems[b]).start()
            for b in range(K):
                sidx(j + b, s_bufs[b], ssems[b]).wait()
                gather(s_bufs[b], r_bufs[b], gsems[b]).start()
            for b in range(K):
                didx(j + b, d_bufs[b], dsems[b]).wait()
            for b in range(K):
                gather(s_bufs[b], r_bufs[b], gsems[b]).wait()
            scats = []
            for b in range(K):
                sc = pltpu.make_async_copy(
                    r_bufs[b], acc_sh.at[d_bufs[b]], gsems[b])
                sc.start(add=True)
                scats.append(sc)
            for sc in scats:
                sc.wait()

    pl.run_scoped(_edge_loop, *([pltpu.SemaphoreType.DMA(())] * (3 * K)))

    plsc.subcore_barrier()

    @pl.loop(0, zrows // CHUNK)
    def _(j):
        ob = s * zrows + j * CHUNK
        pltpu.sync_copy(acc_sh.at[pl.ds(ob, CHUNK)], rows0_v)
        pltpu.sync_copy(rows0_v, out_hbm.at[c, pl.ds(ob, CHUNK)])


def _tc_first(deg_ref, x_ref, w1_ref, g_ref, dinv_ref):
    deg = 1.0 + deg_ref[0, 0:N, 0:1] + deg_ref[1, 0:N, 0:1]
    dinv = lax.rsqrt(deg)
    h = jnp.dot(x_ref[...], w1_ref[...], preferred_element_type=jnp.float32)
    g_ref[pl.ds(0, N), :] = dinv * h
    g_ref[pl.ds(N, NPAD - N), :] = jnp.zeros((NPAD - N, D), jnp.float32)
    dinv_ref[...] = dinv


def _tc_mid(s_ref, g_ref, dinv_ref, b_ref, w2_ref, g2_ref):
    dinv = dinv_ref[...]
    z = dinv * (s_ref[0, 0:N, :] + s_ref[1, 0:N, :] + g_ref[0:N, :]) + b_ref[...]
    z = jnp.maximum(z, 0.0)
    h = jnp.dot(z, w2_ref[...], preferred_element_type=jnp.float32)
    g2_ref[pl.ds(0, N), :] = dinv * h
    g2_ref[pl.ds(N, NPAD - N), :] = jnp.zeros((NPAD - N, D), jnp.float32)


def _tc_head(s_ref, g_ref, dinv_ref, b_ref, batch_ref, wfc_ref, bfc_ref, out_ref):
    dinv = dinv_ref[...]
    z = dinv * (s_ref[0, 0:N, :] + s_ref[1, 0:N, :] + g_ref[0:N, :]) + b_ref[...]
    z = jnp.maximum(z, 0.0)
    seg = lax.broadcasted_iota(jnp.int32, (G, N), 0)
    oh = (seg == batch_ref[...]).astype(jnp.float32)
    psum = jnp.dot(oh, z, preferred_element_type=jnp.float32)
    cnt = jnp.dot(oh, jnp.ones((N, 1), jnp.float32),
                  preferred_element_type=jnp.float32)
    pooled = psum / jnp.maximum(cnt, 1.0)
    out_ref[...] = jnp.dot(pooled, wfc_ref[...],
                           preferred_element_type=jnp.float32) + bfc_ref[...]


def kernel(x, edge_index, batch, W1, b1, W2, b2, Wfc, bfc):
    src = edge_index[0]
    dst = edge_index[1]
    npad_extra = E_PAD2 - E
    pad_src = jnp.full((npad_extra,), N, jnp.int32)
    pad_dst = N + (jnp.arange(npad_extra, dtype=jnp.int32) % (NPAD - N))
    srcp = jnp.concatenate([src, pad_src])
    dstp = jnp.concatenate([dst, pad_dst])
    batch_row = batch.reshape(1, N)

    degp = _sc_degree(dstp).reshape(NC, NPAD, 1)

    g1, dinv = pl.pallas_call(
        _tc_first,
        out_shape=(jax.ShapeDtypeStruct((NPAD, D), jnp.float32),
                   jax.ShapeDtypeStruct((N, 1), jnp.float32)),
    )(degp, x, W1)

    s1 = _sc_segsum(g1, srcp, dstp)

    g2 = pl.pallas_call(
        _tc_mid,
        out_shape=jax.ShapeDtypeStruct((NPAD, D), jnp.float32),
    )(s1, g1, dinv, b1.reshape(1, H), W2)

    s2 = _sc_segsum(g2, srcp, dstp)

    out = pl.pallas_call(
        _tc_head,
        out_shape=jax.ShapeDtypeStruct((G, C), jnp.float32),
    )(s2, g2, dinv, b2.reshape(1, H), batch_row, Wfc, bfc.reshape(1, C))
    return out


# speedup vs baseline: 1.0792x; 1.0037x over previous
import functools

import jax
import jax.numpy as jnp
from jax import lax
from jax.experimental import pallas as pl
from jax.experimental.pallas import tpu as pltpu
from jax.experimental.pallas import tpu_sc as plsc

N = 10000
E = 320000
D = 128
H = 128
C = 10
G = 64

NC = 2
NS = 16
L = 16

CHUNK = 128
K = 2
EPT = 10240
E_PAD = NC * NS * EPT
NCHUNK = EPT // CHUNK
NPAD = 10240
DW = 16
E_PAD2 = E_PAD + 1024

_mesh = plsc.VectorSubcoreMesh(
    core_axis_name="c", subcore_axis_name="s", num_cores=NC, num_subcores=NS)


@functools.partial(
    pl.kernel,
    out_type=jax.ShapeDtypeStruct((NC * NPAD,), jnp.float32),
    mesh=_mesh,
    scratch_types=[
        pltpu.VMEM((CHUNK,), jnp.int32),
        pltpu.VMEM((CHUNK,), jnp.float32),
        pltpu.VMEM_SHARED((NPAD,), jnp.float32),
    ],
)
def _sc_degree(dst_hbm, out_hbm, d0_v, ones_v, acc_sh, ):
    c = lax.axis_index("c")
    s = lax.axis_index("s")

    one16 = jnp.ones((L,), jnp.float32)
    zero16 = jnp.zeros((L,), jnp.float32)
    zrows = NPAD // NS

    @pl.loop(0, CHUNK // L)
    def _(r):
        ones_v[pl.ds(r * L, L)] = zero16

    @pl.loop(0, zrows // CHUNK)
    def _(j):
        pltpu.sync_copy(ones_v, acc_sh.at[pl.ds(s * zrows + j * CHUNK, CHUNK)])

    @pl.loop(0, CHUNK // L)
    def _(r):
        ones_v[pl.ds(r * L, L)] = one16

    base = (c * NS + s) * NCHUNK

    plsc.subcore_barrier()

    @pl.loop(0, NCHUNK)
    def _(j):
        pltpu.sync_copy(dst_hbm.at[pl.ds((base + j) * CHUNK, CHUNK)], d0_v)
        pltpu.sync_copy(ones_v, acc_sh.at[d0_v], add=True)

    plsc.subcore_barrier()

    @pl.loop(0, zrows // CHUNK)
    def _(j):
        rb = s * zrows + j * CHUNK
        pltpu.sync_copy(acc_sh.at[pl.ds(rb, CHUNK)], ones_v)
        pltpu.sync_copy(ones_v, out_hbm.at[pl.ds(c * NPAD + rb, CHUNK)])


@functools.partial(
    pl.kernel,
    out_type=jax.ShapeDtypeStruct((NC, NPAD, D), jnp.float32),
    mesh=_mesh,
    scratch_types=(
        [pltpu.VMEM((CHUNK,), jnp.int32) for _ in range(K)]
        + [pltpu.VMEM((CHUNK,), jnp.int32) for _ in range(K)]
        + [pltpu.VMEM((CHUNK, D), jnp.float32) for _ in range(K)]
        + [pltpu.VMEM_SHARED((NPAD, D), jnp.float32)]
    ),
)
def _sc_segsum(g_hbm, src_hbm, dst_hbm, out_hbm, *scr):
    s_bufs = scr[:K]
    d_bufs = scr[K:2 * K]
    r_bufs = scr[2 * K:3 * K]
    acc_sh = scr[3 * K]
    rows0_v = r_bufs[0]

    c = lax.axis_index("c")
    s = lax.axis_index("s")

    zero16 = jnp.zeros((L,), jnp.float32)

    @pl.loop(0, CHUNK)
    def _(r):
        @pl.loop(0, D // L)
        def _(t):
            rows0_v[r, pl.ds(t * L, L)] = zero16

    zrows = NPAD // NS

    @pl.loop(0, zrows // CHUNK)
    def _(j):
        pltpu.sync_copy(rows0_v, acc_sh.at[pl.ds(s * zrows + j * CHUNK, CHUNK)])

    base = (c * NS + s) * NCHUNK

    def sidx(chunk, buf, sem):
        return pltpu.make_async_copy(
            src_hbm.at[pl.ds((base + chunk) * CHUNK, CHUNK)], buf, sem)

    def didx(chunk, buf, sem):
        return pltpu.make_async_copy(
            dst_hbm.at[pl.ds((base + chunk) * CHUNK, CHUNK)], buf, sem)

    def gather(ibuf, rbuf, sem):
        return pltpu.make_async_copy(g_hbm.at[ibuf], rbuf, sem)

    plsc.subcore_barrier()

    def _edge_loop(*sems):
        gsems = sems[:K]
        ssems = sems[K:2 * K]
        dsems = sems[2 * K:]

        @pl.loop(0, NCHUNK, step=K)
        def _(j):
            for b in range(K):
                sidx(j + b, s_bufs[b], ssems[b]).start()
                didx(j + b, d_bufs[b], dsems[b]).start()
            for b in range(K):
                sidx(j + b, s_bufs[b], ssems[b]).wait()
                gather(s_bufs[b], r_bufs[b], gsems[b]).start()
            for b in range(K):
                didx(j + b, d_bufs[b], dsems[b]).wait()
            for b in range(K):
                gather(s_bufs[b], r_bufs[b], gsems[b]).wait()
            scats = []
            for b in range(K):
                sc = pltpu.make_async_copy(
                    r_bufs[b], acc_sh.at[d_bufs[b]], gsems[b])
                sc.start(add=True)
                scats.append(sc)
            for sc in scats:
                sc.wait()

    pl.run_scoped(_edge_loop, *([pltpu.SemaphoreType.DMA(())] * (3 * K)))

    plsc.subcore_barrier()

    @pl.loop(0, zrows // CHUNK)
    def _(j):
        ob = s * zrows + j * CHUNK
        pltpu.sync_copy(acc_sh.at[pl.ds(ob, CHUNK)], rows0_v)
        pltpu.sync_copy(rows0_v, out_hbm.at[c, pl.ds(ob, CHUNK)])


def _tc_first(deg_ref, x_ref, w1_ref, g_ref, dinv_ref):
    deg = 1.0 + deg_ref[0, 0:N, 0:1] + deg_ref[1, 0:N, 0:1]
    dinv = lax.rsqrt(deg)
    h = jnp.dot(x_ref[...], w1_ref[...], preferred_element_type=jnp.float32)
    g_ref[pl.ds(0, N), :] = dinv * h
    g_ref[pl.ds(N, NPAD - N), :] = jnp.zeros((NPAD - N, D), jnp.float32)
    dinv_ref[...] = dinv


def _tc_mid(s_ref, g_ref, dinv_ref, b_ref, w2_ref, g2_ref):
    dinv = dinv_ref[...]
    z = dinv * (s_ref[0, 0:N, :] + s_ref[1, 0:N, :] + g_ref[0:N, :]) + b_ref[...]
    z = jnp.maximum(z, 0.0)
    h = jnp.dot(z, w2_ref[...], preferred_element_type=jnp.float32)
    g2_ref[pl.ds(0, N), :] = dinv * h
    g2_ref[pl.ds(N, NPAD - N), :] = jnp.zeros((NPAD - N, D), jnp.float32)


def _tc_head(s_ref, g_ref, dinv_ref, b_ref, batch_ref, wfc_ref, bfc_ref, out_ref):
    dinv = dinv_ref[...]
    z = dinv * (s_ref[0, 0:N, :] + s_ref[1, 0:N, :] + g_ref[0:N, :]) + b_ref[...]
    z = jnp.maximum(z, 0.0)
    seg = lax.broadcasted_iota(jnp.int32, (G, N), 0)
    oh = (seg == batch_ref[...]).astype(jnp.float32)
    psum = jnp.dot(oh, z, preferred_element_type=jnp.float32)
    cnt = jnp.dot(oh, jnp.ones((N, 1), jnp.float32),
                  preferred_element_type=jnp.float32)
    pooled = psum / jnp.maximum(cnt, 1.0)
    out_ref[...] = jnp.dot(pooled, wfc_ref[...],
                           preferred_element_type=jnp.float32) + bfc_ref[...]


def kernel(x, edge_index, batch, W1, b1, W2, b2, Wfc, bfc):
    src = edge_index[0]
    dst = edge_index[1]
    npad_extra = E_PAD2 - E
    pad_src = jnp.full((npad_extra,), N, jnp.int32)
    pad_dst = N + (jnp.arange(npad_extra, dtype=jnp.int32) % (NPAD - N))
    srcp = jnp.concatenate([src, pad_src])
    dstp = jnp.concatenate([dst, pad_dst])
    batch_row = batch.reshape(1, N)

    degp = _sc_degree(dstp).reshape(NC, NPAD, 1)

    g1, dinv = pl.pallas_call(
        _tc_first,
        out_shape=(jax.ShapeDtypeStruct((NPAD, D), jnp.float32),
                   jax.ShapeDtypeStruct((N, 1), jnp.float32)),
    )(degp, x, W1)

    s1 = _sc_segsum(g1, srcp, dstp)

    g2 = pl.pallas_call(
        _tc_mid,
        out_shape=jax.ShapeDtypeStruct((NPAD, D), jnp.float32),
    )(s1, g1, dinv, b1.reshape(1, H), W2)

    s2 = _sc_segsum(g2, srcp, dstp)

    out = pl.pallas_call(
        _tc_head,
        out_shape=jax.ShapeDtypeStruct((G, C), jnp.float32),
    )(s2, g2, dinv, b2.reshape(1, H), batch_row, Wfc, bfc.reshape(1, C))
    return out


# speedup vs baseline: 1.0838x; 1.0042x over previous
import functools

import jax
import jax.numpy as jnp
from jax import lax
from jax.experimental import pallas as pl
from jax.experimental.pallas import tpu as pltpu
from jax.experimental.pallas import tpu_sc as plsc

N = 10000
E = 320000
D = 128
H = 128
C = 10
G = 64

NC = 2
NS = 16
L = 16

CHUNK = 128
K = 2
EPT = 10240
E_PAD = NC * NS * EPT
NCHUNK = EPT // CHUNK
NPAD = 10240
DW = 16
E_PAD2 = E_PAD + 1024

_mesh = plsc.VectorSubcoreMesh(
    core_axis_name="c", subcore_axis_name="s", num_cores=NC, num_subcores=NS)


@functools.partial(
    pl.kernel,
    out_type=jax.ShapeDtypeStruct((NC * NPAD,), jnp.float32),
    mesh=_mesh,
    scratch_types=[
        pltpu.VMEM((CHUNK,), jnp.int32),
        pltpu.VMEM((CHUNK,), jnp.float32),
        pltpu.VMEM_SHARED((NPAD,), jnp.float32),
    ],
)
def _sc_degree(dst_hbm, out_hbm, d0_v, ones_v, acc_sh, ):
    c = lax.axis_index("c")
    s = lax.axis_index("s")

    one16 = jnp.ones((L,), jnp.float32)
    zero16 = jnp.zeros((L,), jnp.float32)
    zrows = NPAD // NS

    @pl.loop(0, CHUNK // L)
    def _(r):
        ones_v[pl.ds(r * L, L)] = zero16

    @pl.loop(0, zrows // CHUNK)
    def _(j):
        pltpu.sync_copy(ones_v, acc_sh.at[pl.ds(s * zrows + j * CHUNK, CHUNK)])

    @pl.loop(0, CHUNK // L)
    def _(r):
        ones_v[pl.ds(r * L, L)] = one16

    base = (c * NS + s) * NCHUNK

    plsc.subcore_barrier()

    @pl.loop(0, NCHUNK)
    def _(j):
        pltpu.sync_copy(dst_hbm.at[pl.ds((base + j) * CHUNK, CHUNK)], d0_v)
        pltpu.sync_copy(ones_v, acc_sh.at[d0_v], add=True)

    plsc.subcore_barrier()

    @pl.loop(0, zrows // CHUNK)
    def _(j):
        rb = s * zrows + j * CHUNK
        pltpu.sync_copy(acc_sh.at[pl.ds(rb, CHUNK)], ones_v)
        pltpu.sync_copy(ones_v, out_hbm.at[pl.ds(c * NPAD + rb, CHUNK)])


@functools.partial(
    pl.kernel,
    out_type=jax.ShapeDtypeStruct((NC, NPAD, D), jnp.float32),
    mesh=_mesh,
    scratch_types=(
        [pltpu.VMEM((CHUNK,), jnp.int32) for _ in range(K)]
        + [pltpu.VMEM((CHUNK,), jnp.int32) for _ in range(K)]
        + [pltpu.VMEM((CHUNK, D), jnp.float32) for _ in range(K)]
        + [pltpu.VMEM_SHARED((NPAD, D), jnp.float32)]
    ),
)
def _sc_segsum(g_hbm, src_hbm, dst_hbm, out_hbm, *scr):
    s_bufs = scr[:K]
    d_bufs = scr[K:2 * K]
    r_bufs = scr[2 * K:3 * K]
    acc_sh = scr[3 * K]
    rows0_v = r_bufs[0]

    c = lax.axis_index("c")
    s = lax.axis_index("s")

    zero16 = jnp.zeros((L,), jnp.float32)

    @pl.loop(0, CHUNK)
    def _(r):
        @pl.loop(0, D // L)
        def _(t):
            rows0_v[r, pl.ds(t * L, L)] = zero16

    zrows = NPAD // NS

    @pl.loop(0, zrows // CHUNK)
    def _(j):
        pltpu.sync_copy(rows0_v, acc_sh.at[pl.ds(s * zrows + j * CHUNK, CHUNK)])

    base = (c * NS + s) * NCHUNK

    def sidx(chunk, buf, sem):
        return pltpu.make_async_copy(
            src_hbm.at[pl.ds((base + chunk) * CHUNK, CHUNK)], buf, sem)

    def didx(chunk, buf, sem):
        return pltpu.make_async_copy(
            dst_hbm.at[pl.ds((base + chunk) * CHUNK, CHUNK)], buf, sem)

    def gather(ibuf, rbuf, sem):
        return pltpu.make_async_copy(g_hbm.at[ibuf], rbuf, sem)

    plsc.subcore_barrier()

    def _edge_loop(*sems):
        gsems = sems[:K]
        ssems = sems[K:2 * K]
        dsems = sems[2 * K:]

        @pl.loop(0, NCHUNK, step=K)
        def _(j):
            for b in range(K):
                sidx(j + b, s_bufs[b], ssems[b]).start()
                didx(j + b, d_bufs[b], dsems[b]).start()
            for b in range(K):
                sidx(j + b, s_bufs[b], ssems[b]).wait()
                gather(s_bufs[b], r_bufs[b], gsems[b]).start()
            for b in range(K):
                didx(j + b, d_bufs[b], dsems[b]).wait()
            for b in range(K):
                gather(s_bufs[b], r_bufs[b], gsems[b]).wait()
            scats = []
            for b in range(K):
                sc = pltpu.make_async_copy(
                    r_bufs[b], acc_sh.at[d_bufs[b]], gsems[b])
                sc.start(add=True)
                scats.append(sc)
            for sc in scats:
                sc.wait()

    pl.run_scoped(_edge_loop, *([pltpu.SemaphoreType.DMA(())] * (3 * K)))

    plsc.subcore_barrier()

    ob = s * zrows
    pltpu.sync_copy(acc_sh.at[pl.ds(ob, zrows)], out_hbm.at[c, pl.ds(ob, zrows)])


def _tc_first(deg_ref, x_ref, w1_ref, g_ref, dinv_ref):
    deg = 1.0 + deg_ref[0, 0:N, 0:1] + deg_ref[1, 0:N, 0:1]
    dinv = lax.rsqrt(deg)
    h = jnp.dot(x_ref[...], w1_ref[...], preferred_element_type=jnp.float32)
    g_ref[pl.ds(0, N), :] = dinv * h
    g_ref[pl.ds(N, NPAD - N), :] = jnp.zeros((NPAD - N, D), jnp.float32)
    dinv_ref[...] = dinv


def _tc_mid(s_ref, g_ref, dinv_ref, b_ref, w2_ref, g2_ref):
    dinv = dinv_ref[...]
    z = dinv * (s_ref[0, 0:N, :] + s_ref[1, 0:N, :] + g_ref[0:N, :]) + b_ref[...]
    z = jnp.maximum(z, 0.0)
    h = jnp.dot(z, w2_ref[...], preferred_element_type=jnp.float32)
    g2_ref[pl.ds(0, N), :] = dinv * h
    g2_ref[pl.ds(N, NPAD - N), :] = jnp.zeros((NPAD - N, D), jnp.float32)


def _tc_head(s_ref, g_ref, dinv_ref, b_ref, batch_ref, wfc_ref, bfc_ref, out_ref):
    dinv = dinv_ref[...]
    z = dinv * (s_ref[0, 0:N, :] + s_ref[1, 0:N, :] + g_ref[0:N, :]) + b_ref[...]
    z = jnp.maximum(z, 0.0)
    seg = lax.broadcasted_iota(jnp.int32, (G, N), 0)
    oh = (seg == batch_ref[...]).astype(jnp.float32)
    psum = jnp.dot(oh, z, preferred_element_type=jnp.float32)
    cnt = jnp.dot(oh, jnp.ones((N, 1), jnp.float32),
                  preferred_element_type=jnp.float32)
    pooled = psum / jnp.maximum(cnt, 1.0)
    out_ref[...] = jnp.dot(pooled, wfc_ref[...],
                           preferred_element_type=jnp.float32) + bfc_ref[...]


def kernel(x, edge_index, batch, W1, b1, W2, b2, Wfc, bfc):
    src = edge_index[0]
    dst = edge_index[1]
    npad_extra = E_PAD2 - E
    pad_src = jnp.full((npad_extra,), N, jnp.int32)
    pad_dst = N + (jnp.arange(npad_extra, dtype=jnp.int32) % (NPAD - N))
    srcp = jnp.concatenate([src, pad_src])
    dstp = jnp.concatenate([dst, pad_dst])
    batch_row = batch.reshape(1, N)

    degp = _sc_degree(dstp).reshape(NC, NPAD, 1)

    g1, dinv = pl.pallas_call(
        _tc_first,
        out_shape=(jax.ShapeDtypeStruct((NPAD, D), jnp.float32),
                   jax.ShapeDtypeStruct((N, 1), jnp.float32)),
    )(degp, x, W1)

    s1 = _sc_segsum(g1, srcp, dstp)

    g2 = pl.pallas_call(
        _tc_mid,
        out_shape=jax.ShapeDtypeStruct((NPAD, D), jnp.float32),
    )(s1, g1, dinv, b1.reshape(1, H), W2)

    s2 = _sc_segsum(g2, srcp, dstp)

    out = pl.pallas_call(
        _tc_head,
        out_shape=jax.ShapeDtypeStruct((G, C), jnp.float32),
    )(s2, g2, dinv, b2.reshape(1, H), batch_row, Wfc, bfc.reshape(1, C))
    return out
